# baseline (speedup 1.0000x reference)
import functools

import jax
import jax.numpy as jnp
from jax import lax
from jax.experimental import pallas as pl
from jax.experimental.pallas import tpu as pltpu
from jax.experimental.pallas import tpu_sc as plsc

VOCAB = 256
EMBED_DIM = 32
NUM_CLASSES = 100
BATCH = 16384
SEQ = 200

NC, NS, L = 2, 16, 16
NW = NC * NS
ROWS_PER_W = BATCH // NW
SLICE = 256
NSLICE = ROWS_PER_W // SLICE
GPS = SLICE // L


def _sc_histogram(xT):
    mesh = plsc.VectorSubcoreMesh(
        core_axis_name="c", subcore_axis_name="s",
        num_cores=NC, num_subcores=NS)

    @functools.partial(
        pl.kernel,
        out_type=jax.ShapeDtypeStruct((BATCH, VOCAB), jnp.float32),
        mesh=mesh,
        compiler_params=pltpu.CompilerParams(needs_layout_passes=False),
        scratch_types=[
            pltpu.VMEM((SEQ, SLICE), jnp.int32),
            pltpu.VMEM((SEQ, SLICE), jnp.int32),
            pltpu.VMEM((L, VOCAB), jnp.float32),
            pltpu.VMEM((L, VOCAB), jnp.float32),
            pltpu.SemaphoreType.DMA,
            pltpu.SemaphoreType.DMA,
            pltpu.SemaphoreType.DMA,
            pltpu.SemaphoreType.DMA,
        ],
    )
    def hist_kernel(xT_hbm, out_hbm, x0, x1, h0, h1, sx0, sx1, so0, so1):
        wid = lax.axis_index("s") * NC + lax.axis_index("c")
        lane = lax.iota(jnp.int32, L)
        ones = jnp.full((L,), 1.0, jnp.float32)
        zeros = jnp.zeros((L,), jnp.float32)
        xbufs, xsems = (x0, x1), (sx0, sx1)
        hbufs, hsems = (h0, h1), (so0, so1)
        col_base = wid * ROWS_PER_W

        pending_x = {0: pltpu.async_copy(
            xT_hbm.at[:, pl.ds(col_base, SLICE)], x0, sx0)}
        pending_h = {}
        for c in range(NSLICE):
            xb = xbufs[c % 2]
            pending_x[c].wait()
            if c + 1 < NSLICE:
                pending_x[c + 1] = pltpu.async_copy(
                    xT_hbm.at[:, pl.ds(col_base + (c + 1) * SLICE, SLICE)],
                    xbufs[(c + 1) % 2], xsems[(c + 1) % 2])
            for gg in range(GPS):
                k = c * GPS + gg
                hb = hbufs[k % 2]
                if k >= 2:
                    pending_h[k - 2].wait()

                @plsc.parallel_loop(0, VOCAB // L, 1, unroll=2)
                def zero_body(j):
                    for r in range(L):
                        hb[r, pl.ds(j * L, L)] = zeros

                @plsc.parallel_loop(0, SEQ, 1, unroll=8)
                def scatter_body(s):
                    vals = xb[s, pl.ds(gg * L, L)]
                    plsc.addupdate_scatter(hb, [lane, vals], ones)

                pending_h[k] = pltpu.async_copy(
                    hb, out_hbm.at[pl.ds(col_base + k * L, L), :],
                    hsems[k % 2])
        pending_h[NSLICE * GPS - 2].wait()
        pending_h[NSLICE * GPS - 1].wait()

    return hist_kernel(xT)


def _tc_classify(counts, table, W, b):
    BLK = 8192

    def body(c_ref, t_ref, w_ref, b_ref, o_ref):
        m = jnp.dot(t_ref[...], w_ref[...],
                    preferred_element_type=jnp.float32)
        out = lax.dot_general(
            m, c_ref[...], (((0,), (1,)), ((), ())),
            preferred_element_type=jnp.float32)
        o_ref[...] = out * (1.0 / SEQ) + b_ref[...]

    return pl.pallas_call(
        body,
        grid=(BATCH // BLK,),
        in_specs=[
            pl.BlockSpec((BLK, VOCAB), lambda j: (j, 0)),
            pl.BlockSpec((VOCAB, EMBED_DIM), lambda j: (0, 0)),
            pl.BlockSpec((EMBED_DIM, NUM_CLASSES), lambda j: (0, 0)),
            pl.BlockSpec((NUM_CLASSES, 1), lambda j: (0, 0)),
        ],
        out_specs=pl.BlockSpec((NUM_CLASSES, BLK), lambda j: (0, j)),
        out_shape=jax.ShapeDtypeStruct((NUM_CLASSES, BATCH), jnp.float32),
    )(counts, table, W, b.reshape(NUM_CLASSES, 1))


def kernel(x, table, W, b):
    xT = jnp.transpose(x.astype(jnp.int32))
    counts = _sc_histogram(xT)
    return jnp.transpose(_tc_classify(counts, table, W, b))


# speedup vs baseline: 1.0235x; 1.0235x over previous
import functools

import jax
import jax.numpy as jnp
from jax import lax
from jax.experimental import pallas as pl
from jax.experimental.pallas import tpu as pltpu
from jax.experimental.pallas import tpu_sc as plsc

VOCAB = 256
EMBED_DIM = 32
NUM_CLASSES = 100
BATCH = 16384
SEQ = 200

NC, NS, L = 2, 16, 16
NW = NC * NS
ROWS_PER_W = BATCH // NW
SLICE = 128
NSLICE = ROWS_PER_W // SLICE
GPS = SLICE // L


def _sc_histogram(xT):
    mesh = plsc.VectorSubcoreMesh(
        core_axis_name="c", subcore_axis_name="s",
        num_cores=NC, num_subcores=NS)

    @functools.partial(
        pl.kernel,
        out_type=jax.ShapeDtypeStruct((BATCH, VOCAB), jnp.float32),
        mesh=mesh,
        compiler_params=pltpu.CompilerParams(needs_layout_passes=False),
        scratch_types=[
            pltpu.VMEM((SEQ, SLICE), jnp.int32),
            pltpu.VMEM((SEQ, SLICE), jnp.int32),
            pltpu.VMEM((L, VOCAB), jnp.float32),
            pltpu.VMEM((L, VOCAB), jnp.float32),
            pltpu.SemaphoreType.DMA,
            pltpu.SemaphoreType.DMA,
            pltpu.SemaphoreType.DMA,
            pltpu.SemaphoreType.DMA,
        ],
    )
    def hist_kernel(xT_hbm, out_hbm, x0, x1, h0, h1, sx0, sx1, so0, so1):
        wid = lax.axis_index("s") * NC + lax.axis_index("c")
        lane = lax.iota(jnp.int32, L)
        ones = jnp.full((L,), 1.0, jnp.float32)
        zeros = jnp.zeros((L,), jnp.float32)
        xbufs, xsems = (x0, x1), (sx0, sx1)
        hbufs, hsems = (h0, h1), (so0, so1)
        col_base = wid * ROWS_PER_W

        pending_x = {0: pltpu.async_copy(
            xT_hbm.at[:, pl.ds(col_base, SLICE)], x0, sx0)}
        pending_h = {}
        for c in range(NSLICE):
            xb = xbufs[c % 2]
            pending_x[c].wait()
            if c + 1 < NSLICE:
                pending_x[c + 1] = pltpu.async_copy(
                    xT_hbm.at[:, pl.ds(col_base + (c + 1) * SLICE, SLICE)],
                    xbufs[(c + 1) % 2], xsems[(c + 1) % 2])
            for gg in range(GPS):
                k = c * GPS + gg
                hb = hbufs[k % 2]
                if k >= 2:
                    pending_h[k - 2].wait()

                @plsc.parallel_loop(0, VOCAB // L, 1, unroll=2)
                def zero_body(j):
                    for r in range(L):
                        hb[r, pl.ds(j * L, L)] = zeros

                @plsc.parallel_loop(0, SEQ, 1, unroll=8)
                def scatter_body(s):
                    vals = xb[s, pl.ds(gg * L, L)]
                    plsc.addupdate_scatter(hb, [lane, vals], ones)

                pending_h[k] = pltpu.async_copy(
                    hb, out_hbm.at[pl.ds(col_base + k * L, L), :],
                    hsems[k % 2])
        pending_h[NSLICE * GPS - 2].wait()
        pending_h[NSLICE * GPS - 1].wait()

    return hist_kernel(xT)


def _tc_classify(counts, table, W, b):
    BLK = 8192

    def body(c_ref, t_ref, w_ref, b_ref, o_ref):
        m = jnp.dot(t_ref[...], w_ref[...],
                    preferred_element_type=jnp.float32)
        out = lax.dot_general(
            m, c_ref[...], (((0,), (1,)), ((), ())),
            preferred_element_type=jnp.float32)
        o_ref[...] = out * (1.0 / SEQ) + b_ref[...]

    return pl.pallas_call(
        body,
        grid=(BATCH // BLK,),
        in_specs=[
            pl.BlockSpec((BLK, VOCAB), lambda j: (j, 0)),
            pl.BlockSpec((VOCAB, EMBED_DIM), lambda j: (0, 0)),
            pl.BlockSpec((EMBED_DIM, NUM_CLASSES), lambda j: (0, 0)),
            pl.BlockSpec((NUM_CLASSES, 1), lambda j: (0, 0)),
        ],
        out_specs=pl.BlockSpec((NUM_CLASSES, BLK), lambda j: (0, j)),
        out_shape=jax.ShapeDtypeStruct((NUM_CLASSES, BATCH), jnp.float32),
    )(counts, table, W, b.reshape(NUM_CLASSES, 1))


def kernel(x, table, W, b):
    xT = jnp.transpose(x.astype(jnp.int32))
    counts = _sc_histogram(xT)
    return jnp.transpose(_tc_classify(counts, table, W, b))
